# baseline (speedup 1.0000x reference)
import jax
import jax.numpy as jnp
from jax import lax
from jax.experimental import pallas as pl
from jax.experimental.pallas import tpu as pltpu
from jax.experimental.pallas import tpu_sc as plsc

N = 10000
E = 160000
D = 256
H = 128
R = 200
T = 16384

NC = 2
NS = 16
EB = 32
EG = 16
ECH = 320
EGRP = ECH // EG
ESP = ECH * EB
EPAD = NS * ESP
DEPTH = 4
LAG = 2
NP = 10240
NROWS = NP // NS
WB = 64
TS = T // NS
TB = 128
TCH = TS // TB


def _mesh():
    return plsc.VectorSubcoreMesh(
        core_axis_name="c", subcore_axis_name="s", num_cores=NC,
        num_subcores=NS)


def _rel_body(coeff_ref, bases_ref, w_ref, rw_ref, z_ref, ur_ref):
    r = jnp.dot(coeff_ref[...], bases_ref[...],
                preferred_element_type=jnp.float32)
    for k in range(3):
        zk = jnp.dot(r, w_ref[k], preferred_element_type=jnp.float32)
        z_ref[0, k * R:(k + 1) * R, :] = zk[:, :H]
        z_ref[1, k * R:(k + 1) * R, :] = zk[:, H:]
    ur = jnp.dot(r, rw_ref[...], preferred_element_type=jnp.float32)
    ur_ref[0] = ur[:, :H]
    ur_ref[1] = ur[:, H:]


def _rel_tables(coefficients, bases, weights, relation_weight):
    out = jax.ShapeDtypeStruct
    z, ur = pl.pallas_call(
        _rel_body,
        out_shape=(out((NC, 3 * R, H), jnp.float32),
                   out((NC, R, H), jnp.float32)),
    )(coefficients, bases, weights, relation_weight)
    return z.reshape(NC * 3 * R, H), ur.reshape(NC * R, H)


BN = 1000


def _y_body(x_ref, w_ref, y_ref):
    y = jnp.dot(x_ref[...], w_ref[0], preferred_element_type=jnp.float32)
    y_ref[0, 0] = y[:, :H]
    y_ref[1, 0] = y[:, H:]


def _y_tables(entity_embeds, weights):
    y = pl.pallas_call(
        _y_body,
        grid=(3, N // BN),
        in_specs=[
            pl.BlockSpec((BN, D), lambda k, i: (i, 0)),
            pl.BlockSpec((1, D, D), lambda k, i: (k, 0, 0)),
        ],
        out_specs=pl.BlockSpec((NC, 1, BN, H), lambda k, i: (0, k, i, 0)),
        out_shape=jax.ShapeDtypeStruct((NC, 3, N, H), jnp.float32),
    )(entity_embeds, weights)
    return y.reshape(NC * 3 * N, H)


def _agg_body(y_tab, z_tab, gy, gz, sd, zrows, x_out,
              gy_v, gz_v, sd_v,
              ry0, ry1, ry2, ry3, rz0, rz1, rz2, rz3, wb_v, acc_sh,
              sgy0, sgy1, sgy2, sgy3, sgz0, sgz1, sgz2, sgz3,
              ssy0, ssy1, ssy2, ssy3, ssz0, ssz1, ssz2, ssz3):
    c = lax.axis_index("c")
    s = lax.axis_index("s")
    rowy = (ry0, ry1, ry2, ry3)
    rowz = (rz0, rz1, rz2, rz3)
    sem_gy = (sgy0, sgy1, sgy2, sgy3)
    sem_gz = (sgz0, sgz1, sgz2, sgz3)
    sem_sy = (ssy0, ssy1, ssy2, ssy3)
    sem_sz = (ssz0, ssz1, ssz2, ssz3)
    pltpu.sync_copy(zrows, acc_sh.at[pl.ds(s * NROWS, NROWS)])
    plsc.subcore_barrier()

    def group(g, carry):
        pltpu.sync_copy(gy.at[(c * NS + s) * EGRP + g], gy_v)
        pltpu.sync_copy(gz.at[(c * NS + s) * EGRP + g], gz_v)
        pltpu.sync_copy(sd.at[s * EGRP + g], sd_v)

        gyd = [None] * DEPTH
        gzd = [None] * DEPTH
        syd = [None] * DEPTH
        szd = [None] * DEPTH

        def scat(j):
            pb = j % DEPTH
            gyd[pb].wait()
            gzd[pb].wait()
            ry, rz = rowy[pb], rowz[pb]

            def arow(i, carry2):
                for jj in range(H // 16):
                    sl = pl.ds(jj * 16, 16)
                    ry[i, sl] = ry[i, sl] + rz[i, sl]
                return carry2
            lax.fori_loop(0, EB, arow, 0)
            syd[pb] = pltpu.async_copy(
                rowy[pb], acc_sh.at[sd_v.at[j]], sem_sy[pb], add=True)

        for j in range(EG):
            b = j % DEPTH
            if j >= DEPTH:
                syd[b].wait()
            gyd[b] = pltpu.async_copy(y_tab.at[gy_v.at[j]], rowy[b],
                                      sem_gy[b])
            gzd[b] = pltpu.async_copy(z_tab.at[gz_v.at[j]], rowz[b],
                                      sem_gz[b])
            if j >= LAG:
                scat(j - LAG)
        for j in range(EG - LAG, EG):
            scat(j)
        for b in range(DEPTH):
            syd[b].wait()
        return carry
    lax.fori_loop(0, EGRP, group, 0)
    plsc.subcore_barrier()

    def wb_step(j, carry1):
        base = s * NROWS + j * WB
        pltpu.sync_copy(acc_sh.at[pl.ds(base, WB)], wb_v)

        def row(i, carry2):
            for jj in range(H // 16):
                sl = pl.ds(jj * 16, 16)
                v = wb_v[i, sl]
                wb_v[i, sl] = (jnp.maximum(v, 0.0) +
                               0.01 * jnp.minimum(v, 0.0))
            return carry2
        lax.fori_loop(0, WB, row, 0)
        pltpu.sync_copy(wb_v, x_out.at[pl.ds(c * NP + base, WB)])
        return carry1
    lax.fori_loop(0, NROWS // WB, wb_step, 0)


def _aggregate(y_tab, z_tab, gy, gz, sd):
    zrows = jnp.zeros((NROWS, H), jnp.float32)
    f = pl.kernel(
        _agg_body,
        out_type=jax.ShapeDtypeStruct((NC * NP, H), jnp.float32),
        mesh=_mesh(),
        scratch_types=(
            [pltpu.VMEM((EG, EB), jnp.int32)] * 3 +
            [pltpu.VMEM((EB, H), jnp.float32)] * (2 * DEPTH) +
            [pltpu.VMEM((WB, H), jnp.float32),
             pltpu.VMEM_SHARED((NP, H), jnp.float32)] +
            [pltpu.SemaphoreType.DMA] * (4 * DEPTH)
        ),
    )
    return f(y_tab, z_tab, gy, gz, sd, zrows)


def _score_body(x_tab, ur_tab, hi, ri, ti, d_out,
                hi_v, ri_v, ti_v, h_v, r_v, t_v, sem):
    c = lax.axis_index("c")
    s = lax.axis_index("s")
    pltpu.sync_copy(hi.at[c * NS + s], hi_v)
    pltpu.sync_copy(ri.at[c * NS + s], ri_v)
    pltpu.sync_copy(ti.at[c * NS + s], ti_v)

    def chunk(k, carry):
        pltpu.async_copy(x_tab.at[hi_v.at[k]], h_v, sem).wait()
        pltpu.async_copy(ur_tab.at[ri_v.at[k]], r_v, sem).wait()
        pltpu.async_copy(x_tab.at[ti_v.at[k]], t_v, sem).wait()

        def row(i, carry2):
            for jj in range(H // 16):
                sl = pl.ds(jj * 16, 16)
                h_v[i, sl] = h_v[i, sl] + r_v[i, sl] - t_v[i, sl]
            return carry2
        lax.fori_loop(0, TB, row, 0)
        pltpu.sync_copy(h_v, d_out.at[pl.ds(c * T + s * TS + k * TB, TB)])
        return carry
    lax.fori_loop(0, TCH, chunk, 0)


def _score_gather(x_tab, ur_tab, hi, ri, ti):
    f = pl.kernel(
        _score_body,
        out_type=jax.ShapeDtypeStruct((NC * T, H), jnp.float32),
        mesh=_mesh(),
        scratch_types=[
            pltpu.VMEM((TCH, TB), jnp.int32),
            pltpu.VMEM((TCH, TB), jnp.int32),
            pltpu.VMEM((TCH, TB), jnp.int32),
            pltpu.VMEM((TB, H), jnp.float32),
            pltpu.VMEM((TB, H), jnp.float32),
            pltpu.VMEM((TB, H), jnp.float32),
            pltpu.SemaphoreType.DMA,
        ],
    )
    return f(x_tab, ur_tab, hi, ri, ti)


BT = 2048


def _fin_body(d0_ref, d1_ref, o_ref):
    ssum = (jnp.sum(jnp.abs(d0_ref[...]), axis=1) +
            jnp.sum(jnp.abs(d1_ref[...]), axis=1))
    o_ref[...] = jax.nn.sigmoid(ssum)


def _finish(d_all):
    return pl.pallas_call(
        _fin_body,
        grid=(T // BT,),
        in_specs=[pl.BlockSpec((BT, H), lambda i: (i, 0)),
                  pl.BlockSpec((BT, H), lambda i: (i + T // BT, 0))],
        out_specs=pl.BlockSpec((BT,), lambda i: (i,)),
        out_shape=jax.ShapeDtypeStruct((T,), jnp.float32),
    )(d_all, d_all)


def kernel(entity_embeds, bases, coefficients, weights, relation_weight,
           edge_index, edge_type, masks, triple_batch):
    src = edge_index[0]
    dst = edge_index[1]
    npad_e = EPAD - E
    coff = jnp.arange(NC, dtype=jnp.int32)[:, None]
    gy = jnp.pad((masks * N + src).astype(jnp.int32), (0, npad_e))
    gy = (gy[None, :] + coff * (3 * N)).reshape(NC * NS * EGRP, EG, EB)
    gz = jnp.pad((masks * R + edge_type).astype(jnp.int32), (0, npad_e))
    gz = (gz[None, :] + coff * (3 * R)).reshape(NC * NS * EGRP, EG, EB)
    pad_sink = N + (jnp.arange(npad_e, dtype=jnp.int32) & (NP - N - 1))
    sd = jnp.concatenate([dst.astype(jnp.int32), pad_sink])
    sd = sd.reshape(NS * EGRP, EG, EB)
    hi = triple_batch[:, 0].astype(jnp.int32)
    hi = (hi[None, :] + coff * NP).reshape(NC * NS, TCH, TB)
    ri = triple_batch[:, 1].astype(jnp.int32)
    ri = (ri[None, :] + coff * R).reshape(NC * NS, TCH, TB)
    ti = triple_batch[:, 2].astype(jnp.int32)
    ti = (ti[None, :] + coff * NP).reshape(NC * NS, TCH, TB)

    z_tab, ur_tab = _rel_tables(coefficients, bases, weights, relation_weight)
    y_tab = _y_tables(entity_embeds, weights)
    x_all = _aggregate(y_tab, z_tab, gy, gz, sd)
    d_all = _score_gather(x_all, ur_tab, hi, ri, ti)
    return _finish(d_all)


# speedup vs baseline: 1.0033x; 1.0033x over previous
import jax
import jax.numpy as jnp
from jax import lax
from jax.experimental import pallas as pl
from jax.experimental.pallas import tpu as pltpu
from jax.experimental.pallas import tpu_sc as plsc

N = 10000
E = 160000
D = 256
H = 128
R = 200
T = 16384

NC = 2
NS = 16
EB = 32
EG = 16
ECH = 320
EGRP = ECH // EG
ESP = ECH * EB
EPAD = NS * ESP
DEPTH = 4
LAG = 2
NP = 10240
NROWS = NP // NS
WB = 64
TS = T // NS
TB = 128
TCH = TS // TB


def _mesh():
    return plsc.VectorSubcoreMesh(
        core_axis_name="c", subcore_axis_name="s", num_cores=NC,
        num_subcores=NS)


def _rel_body(coeff_ref, bases_ref, w_ref, rw_ref, z_ref, ur_ref):
    r = jnp.dot(coeff_ref[...], bases_ref[...],
                preferred_element_type=jnp.float32)
    for k in range(3):
        zk = jnp.dot(r, w_ref[k], preferred_element_type=jnp.float32)
        z_ref[0, k * R:(k + 1) * R, :] = zk[:, :H]
        z_ref[1, k * R:(k + 1) * R, :] = zk[:, H:]
    ur = jnp.dot(r, rw_ref[...], preferred_element_type=jnp.float32)
    ur_ref[0] = ur[:, :H]
    ur_ref[1] = ur[:, H:]


def _rel_tables(coefficients, bases, weights, relation_weight):
    out = jax.ShapeDtypeStruct
    z, ur = pl.pallas_call(
        _rel_body,
        out_shape=(out((NC, 3 * R, H), jnp.float32),
                   out((NC, R, H), jnp.float32)),
    )(coefficients, bases, weights, relation_weight)
    return z.reshape(NC * 3 * R, H), ur.reshape(NC * R, H)


BN = 1000


def _y_body(x_ref, w_ref, y_ref):
    y = jnp.dot(x_ref[...], w_ref[0], preferred_element_type=jnp.float32)
    y_ref[0, 0] = y[:, :H]
    y_ref[1, 0] = y[:, H:]


def _y_tables(entity_embeds, weights):
    y = pl.pallas_call(
        _y_body,
        grid=(3, N // BN),
        in_specs=[
            pl.BlockSpec((BN, D), lambda k, i: (i, 0)),
            pl.BlockSpec((1, D, D), lambda k, i: (k, 0, 0)),
        ],
        out_specs=pl.BlockSpec((NC, 1, BN, H), lambda k, i: (0, k, i, 0)),
        out_shape=jax.ShapeDtypeStruct((NC, 3, N, H), jnp.float32),
    )(entity_embeds, weights)
    return y.reshape(NC * 3 * N, H)


def _agg_body(y_tab, z_tab, gy, gz, sd, zrows, x_out,
              gy_v, gz_v, sd_v,
              ry0, ry1, ry2, ry3, rz0, rz1, rz2, rz3, wb_v, acc_sh,
              sgy0, sgy1, sgy2, sgy3, sgz0, sgz1, sgz2, sgz3,
              ssy0, ssy1, ssy2, ssy3, ssz0, ssz1, ssz2, ssz3):
    c = lax.axis_index("c")
    s = lax.axis_index("s")
    rowy = (ry0, ry1, ry2, ry3)
    rowz = (rz0, rz1, rz2, rz3)
    sem_gy = (sgy0, sgy1, sgy2, sgy3)
    sem_gz = (sgz0, sgz1, sgz2, sgz3)
    sem_sy = (ssy0, ssy1, ssy2, ssy3)
    sem_sz = (ssz0, ssz1, ssz2, ssz3)
    pltpu.sync_copy(zrows, acc_sh.at[pl.ds(s * NROWS, NROWS)])
    plsc.subcore_barrier()

    def group(g, carry):
        pltpu.sync_copy(gy.at[(c * NS + s) * EGRP + g], gy_v)
        pltpu.sync_copy(gz.at[(c * NS + s) * EGRP + g], gz_v)
        pltpu.sync_copy(sd.at[s * EGRP + g], sd_v)

        gyd = [None] * DEPTH
        gzd = [None] * DEPTH
        syd = [None] * DEPTH
        szd = [None] * DEPTH

        def scat(j):
            pb = j % DEPTH
            gyd[pb].wait()
            syd[pb] = pltpu.async_copy(
                rowy[pb], acc_sh.at[sd_v.at[j]], sem_sy[pb], add=True)
            gzd[pb].wait()
            szd[pb] = pltpu.async_copy(
                rowz[pb], acc_sh.at[sd_v.at[j]], sem_sz[pb], add=True)

        for j in range(EG):
            b = j % DEPTH
            if j >= DEPTH:
                syd[b].wait()
                szd[b].wait()
            gyd[b] = pltpu.async_copy(y_tab.at[gy_v.at[j]], rowy[b],
                                      sem_gy[b])
            gzd[b] = pltpu.async_copy(z_tab.at[gz_v.at[j]], rowz[b],
                                      sem_gz[b])
            if j >= LAG:
                scat(j - LAG)
        for j in range(EG - LAG, EG):
            scat(j)
        for b in range(DEPTH):
            syd[b].wait()
            szd[b].wait()
        return carry
    lax.fori_loop(0, EGRP, group, 0)
    plsc.subcore_barrier()

    def wb_step(j, carry1):
        base = s * NROWS + j * WB
        pltpu.sync_copy(acc_sh.at[pl.ds(base, WB)], wb_v)

        def row(i, carry2):
            for jj in range(H // 16):
                sl = pl.ds(jj * 16, 16)
                v = wb_v[i, sl]
                wb_v[i, sl] = (jnp.maximum(v, 0.0) +
                               0.01 * jnp.minimum(v, 0.0))
            return carry2
        lax.fori_loop(0, WB, row, 0)
        pltpu.sync_copy(wb_v, x_out.at[pl.ds(c * NP + base, WB)])
        return carry1
    lax.fori_loop(0, NROWS // WB, wb_step, 0)


def _aggregate(y_tab, z_tab, gy, gz, sd):
    zrows = jnp.zeros((NROWS, H), jnp.float32)
    f = pl.kernel(
        _agg_body,
        out_type=jax.ShapeDtypeStruct((NC * NP, H), jnp.float32),
        mesh=_mesh(),
        scratch_types=(
            [pltpu.VMEM((EG, EB), jnp.int32)] * 3 +
            [pltpu.VMEM((EB, H), jnp.float32)] * (2 * DEPTH) +
            [pltpu.VMEM((WB, H), jnp.float32),
             pltpu.VMEM_SHARED((NP, H), jnp.float32)] +
            [pltpu.SemaphoreType.DMA] * (4 * DEPTH)
        ),
    )
    return f(y_tab, z_tab, gy, gz, sd, zrows)


def _score_body(x_tab, ur_tab, hi, ri, ti, d_out,
                hi_v, ri_v, ti_v, h_v, r_v, t_v, sem):
    c = lax.axis_index("c")
    s = lax.axis_index("s")
    pltpu.sync_copy(hi.at[c * NS + s], hi_v)
    pltpu.sync_copy(ri.at[c * NS + s], ri_v)
    pltpu.sync_copy(ti.at[c * NS + s], ti_v)

    def chunk(k, carry):
        pltpu.async_copy(x_tab.at[hi_v.at[k]], h_v, sem).wait()
        pltpu.async_copy(ur_tab.at[ri_v.at[k]], r_v, sem).wait()
        pltpu.async_copy(x_tab.at[ti_v.at[k]], t_v, sem).wait()

        def row(i, carry2):
            for jj in range(H // 16):
                sl = pl.ds(jj * 16, 16)
                h_v[i, sl] = h_v[i, sl] + r_v[i, sl] - t_v[i, sl]
            return carry2
        lax.fori_loop(0, TB, row, 0)
        pltpu.sync_copy(h_v, d_out.at[pl.ds(c * T + s * TS + k * TB, TB)])
        return carry
    lax.fori_loop(0, TCH, chunk, 0)


def _score_gather(x_tab, ur_tab, hi, ri, ti):
    f = pl.kernel(
        _score_body,
        out_type=jax.ShapeDtypeStruct((NC * T, H), jnp.float32),
        mesh=_mesh(),
        scratch_types=[
            pltpu.VMEM((TCH, TB), jnp.int32),
            pltpu.VMEM((TCH, TB), jnp.int32),
            pltpu.VMEM((TCH, TB), jnp.int32),
            pltpu.VMEM((TB, H), jnp.float32),
            pltpu.VMEM((TB, H), jnp.float32),
            pltpu.VMEM((TB, H), jnp.float32),
            pltpu.SemaphoreType.DMA,
        ],
    )
    return f(x_tab, ur_tab, hi, ri, ti)


BT = 2048


def _fin_body(d0_ref, d1_ref, o_ref):
    ssum = (jnp.sum(jnp.abs(d0_ref[...]), axis=1) +
            jnp.sum(jnp.abs(d1_ref[...]), axis=1))
    o_ref[...] = jax.nn.sigmoid(ssum)


def _finish(d_all):
    return pl.pallas_call(
        _fin_body,
        grid=(T // BT,),
        in_specs=[pl.BlockSpec((BT, H), lambda i: (i, 0)),
                  pl.BlockSpec((BT, H), lambda i: (i + T // BT, 0))],
        out_specs=pl.BlockSpec((BT,), lambda i: (i,)),
        out_shape=jax.ShapeDtypeStruct((T,), jnp.float32),
    )(d_all, d_all)


def kernel(entity_embeds, bases, coefficients, weights, relation_weight,
           edge_index, edge_type, masks, triple_batch):
    src = edge_index[0]
    dst = edge_index[1]
    npad_e = EPAD - E
    coff = jnp.arange(NC, dtype=jnp.int32)[:, None]
    gy = jnp.pad((masks * N + src).astype(jnp.int32), (0, npad_e))
    gy = (gy[None, :] + coff * (3 * N)).reshape(NC * NS * EGRP, EG, EB)
    gz = jnp.pad((masks * R + edge_type).astype(jnp.int32), (0, npad_e))
    gz = (gz[None, :] + coff * (3 * R)).reshape(NC * NS * EGRP, EG, EB)
    pad_sink = N + (jnp.arange(npad_e, dtype=jnp.int32) & (NP - N - 1))
    sd = jnp.concatenate([dst.astype(jnp.int32), pad_sink])
    sd = sd.reshape(NS * EGRP, EG, EB)
    hi = triple_batch[:, 0].astype(jnp.int32)
    hi = (hi[None, :] + coff * NP).reshape(NC * NS, TCH, TB)
    ri = triple_batch[:, 1].astype(jnp.int32)
    ri = (ri[None, :] + coff * R).reshape(NC * NS, TCH, TB)
    ti = triple_batch[:, 2].astype(jnp.int32)
    ti = (ti[None, :] + coff * NP).reshape(NC * NS, TCH, TB)

    z_tab, ur_tab = _rel_tables(coefficients, bases, weights, relation_weight)
    y_tab = _y_tables(entity_embeds, weights)
    x_all = _aggregate(y_tab, z_tab, gy, gz, sd)
    d_all = _score_gather(x_all, ur_tab, hi, ri, ti)
    return _finish(d_all)


# speedup vs baseline: 1.0259x; 1.0225x over previous
import jax
import jax.numpy as jnp
from jax import lax
from jax.experimental import pallas as pl
from jax.experimental.pallas import tpu as pltpu
from jax.experimental.pallas import tpu_sc as plsc

N = 10000
E = 160000
D = 256
H = 128
R = 200
T = 16384

NC = 2
NS = 16
EB = 32
EG = 16
ECH = 320
EGRP = ECH // EG
ESP = ECH * EB
EPAD = NS * ESP
DEPTH = 4
LAG = 2
NP = 10240
NROWS = NP // NS
WB = 64
TS = T // NS
TB = 128
TCH = TS // TB


def _mesh():
    return plsc.VectorSubcoreMesh(
        core_axis_name="c", subcore_axis_name="s", num_cores=NC,
        num_subcores=NS)


def _rel_body(coeff_ref, bases_ref, w_ref, rw_ref, z_ref, ur_ref):
    r = jnp.dot(coeff_ref[...], bases_ref[...],
                preferred_element_type=jnp.float32)
    for k in range(3):
        zk = jnp.dot(r, w_ref[k], preferred_element_type=jnp.float32)
        z_ref[0, k * R:(k + 1) * R, :] = zk[:, :H]
        z_ref[1, k * R:(k + 1) * R, :] = zk[:, H:]
    ur = jnp.dot(r, rw_ref[...], preferred_element_type=jnp.float32)
    ur_ref[0] = ur[:, :H]
    ur_ref[1] = ur[:, H:]


def _rel_tables(coefficients, bases, weights, relation_weight):
    out = jax.ShapeDtypeStruct
    z, ur = pl.pallas_call(
        _rel_body,
        out_shape=(out((NC, 3 * R, H), jnp.float32),
                   out((NC, R, H), jnp.float32)),
    )(coefficients, bases, weights, relation_weight)
    return z.reshape(NC * 3 * R, H), ur.reshape(NC * R, H)


BN = 1000


def _y_body(x_ref, w_ref, y_ref):
    y = jnp.dot(x_ref[...], w_ref[0], preferred_element_type=jnp.float32)
    y_ref[0, 0] = y[:, :H]
    y_ref[1, 0] = y[:, H:]


def _y_tables(entity_embeds, weights):
    y = pl.pallas_call(
        _y_body,
        grid=(3, N // BN),
        in_specs=[
            pl.BlockSpec((BN, D), lambda k, i: (i, 0)),
            pl.BlockSpec((1, D, D), lambda k, i: (k, 0, 0)),
        ],
        out_specs=pl.BlockSpec((NC, 1, BN, H), lambda k, i: (0, k, i, 0)),
        out_shape=jax.ShapeDtypeStruct((NC, 3, N, H), jnp.float32),
    )(entity_embeds, weights)
    return y.reshape(NC * 3 * N, H)


def _agg_body(y_tab, z_tab, gy, gz, sd, zrows, x_out,
              gy_v, gz_v, sd_v,
              ry0, ry1, ry2, ry3, rz0, rz1, rz2, rz3, wb_v, acc_sh,
              sgy0, sgy1, sgy2, sgy3, sgz0, sgz1, sgz2, sgz3,
              ssy0, ssy1, ssy2, ssy3, ssz0, ssz1, ssz2, ssz3):
    c = lax.axis_index("c")
    s = lax.axis_index("s")
    rowy = (ry0, ry1, ry2, ry3)
    rowz = (rz0, rz1, rz2, rz3)
    sem_gy = (sgy0, sgy1, sgy2, sgy3)
    sem_gz = (sgz0, sgz1, sgz2, sgz3)
    sem_sy = (ssy0, ssy1, ssy2, ssy3)
    sem_sz = (ssz0, ssz1, ssz2, ssz3)
    pltpu.sync_copy(zrows, acc_sh.at[pl.ds(s * NROWS, NROWS)])
    plsc.subcore_barrier()

    def group(g, carry):
        pltpu.sync_copy(gy.at[(c * NS + s) * EGRP + g], gy_v)
        pltpu.sync_copy(gz.at[(c * NS + s) * EGRP + g], gz_v)
        pltpu.sync_copy(sd.at[s * EGRP + g], sd_v)

        gyd = [None] * DEPTH
        gzd = [None] * DEPTH
        syd = [None] * DEPTH
        szd = [None] * DEPTH

        def scat(j):
            pb = j % DEPTH
            gyd[pb].wait()
            syd[pb] = pltpu.async_copy(
                rowy[pb], acc_sh.at[sd_v.at[j]], sem_sy[pb], add=True)
            gzd[pb].wait()
            szd[pb] = pltpu.async_copy(
                rowz[pb], acc_sh.at[sd_v.at[j]], sem_sz[pb], add=True)

        for j in range(EG):
            b = j % DEPTH
            if j >= DEPTH:
                syd[b].wait()
                szd[b].wait()
            gyd[b] = pltpu.async_copy(y_tab.at[gy_v.at[j]], rowy[b],
                                      sem_gy[b])
            gzd[b] = pltpu.async_copy(z_tab.at[gz_v.at[j]], rowz[b],
                                      sem_gz[b])
            if j >= LAG:
                scat(j - LAG)
        for j in range(EG - LAG, EG):
            scat(j)
        for b in range(DEPTH):
            syd[b].wait()
            szd[b].wait()
        return carry
    lax.fori_loop(0, EGRP, group, 0)
    plsc.subcore_barrier()

    def wb_step(j, carry1):
        base = s * NROWS + j * WB
        pltpu.sync_copy(acc_sh.at[pl.ds(base, WB)], wb_v)

        def row(i, carry2):
            for jj in range(H // 16):
                sl = pl.ds(jj * 16, 16)
                v = wb_v[i, sl]
                wb_v[i, sl] = (jnp.maximum(v, 0.0) +
                               0.01 * jnp.minimum(v, 0.0))
            return carry2
        lax.fori_loop(0, WB, row, 0)
        pltpu.sync_copy(wb_v, x_out.at[pl.ds(c * NP + base, WB)])
        return carry1
    lax.fori_loop(0, NROWS // WB, wb_step, 0)


def _aggregate(y_tab, z_tab, gy, gz, sd):
    zrows = jnp.zeros((NROWS, H), jnp.float32)
    f = pl.kernel(
        _agg_body,
        out_type=jax.ShapeDtypeStruct((NC * NP, H), jnp.float32),
        mesh=_mesh(),
        scratch_types=(
            [pltpu.VMEM((EG, EB), jnp.int32)] * 3 +
            [pltpu.VMEM((EB, H), jnp.float32)] * (2 * DEPTH) +
            [pltpu.VMEM((WB, H), jnp.float32),
             pltpu.VMEM_SHARED((NP, H), jnp.float32)] +
            [pltpu.SemaphoreType.DMA] * (4 * DEPTH)
        ),
    )
    return f(y_tab, z_tab, gy, gz, sd, zrows)


def _score_body(x_tab, ur_tab, hi, ri, ti, d_out,
                hi_v, ri_v, ti_v, h0, r0, t0, h1, r1, t1,
                sh0, sr0, st0, sh1, sr1, st1, sw0, sw1):
    c = lax.axis_index("c")
    s = lax.axis_index("s")
    pltpu.sync_copy(hi.at[c * NS + s], hi_v)
    pltpu.sync_copy(ri.at[c * NS + s], ri_v)
    pltpu.sync_copy(ti.at[c * NS + s], ti_v)

    hb = (h0, h1)
    rb = (r0, r1)
    tb = (t0, t1)
    sem_h = (sh0, sh1)
    sem_r = (sr0, sr1)
    sem_t = (st0, st1)
    sem_w = (sw0, sw1)
    gd = [[None] * 3, [None] * 3]
    wd = [None, None]

    def emit(k):
        pb = k & 1
        for d_ in gd[pb]:
            d_.wait()
        hv, rv, tv = hb[pb], rb[pb], tb[pb]

        def row(i, carry2):
            for jj in range(H // 16):
                sl = pl.ds(jj * 16, 16)
                hv[i, sl] = hv[i, sl] + rv[i, sl] - tv[i, sl]
            return carry2
        lax.fori_loop(0, TB, row, 0)
        wd[pb] = pltpu.async_copy(
            hv, d_out.at[pl.ds(c * T + s * TS + k * TB, TB)], sem_w[pb])

    for k in range(TCH):
        b = k & 1
        if k >= 2:
            wd[b].wait()
        gd[b] = [
            pltpu.async_copy(x_tab.at[hi_v.at[k]], hb[b], sem_h[b]),
            pltpu.async_copy(ur_tab.at[ri_v.at[k]], rb[b], sem_r[b]),
            pltpu.async_copy(x_tab.at[ti_v.at[k]], tb[b], sem_t[b]),
        ]
        if k >= 1:
            emit(k - 1)
    emit(TCH - 1)
    for b in range(2):
        wd[b].wait()


def _score_gather(x_tab, ur_tab, hi, ri, ti):
    f = pl.kernel(
        _score_body,
        out_type=jax.ShapeDtypeStruct((NC * T, H), jnp.float32),
        mesh=_mesh(),
        scratch_types=(
            [pltpu.VMEM((TCH, TB), jnp.int32)] * 3 +
            [pltpu.VMEM((TB, H), jnp.float32)] * 6 +
            [pltpu.SemaphoreType.DMA] * 8
        ),
    )
    return f(x_tab, ur_tab, hi, ri, ti)


BT = 2048


def _fin_body(d0_ref, d1_ref, o_ref):
    ssum = (jnp.sum(jnp.abs(d0_ref[...]), axis=1) +
            jnp.sum(jnp.abs(d1_ref[...]), axis=1))
    o_ref[...] = jax.nn.sigmoid(ssum)


def _finish(d_all):
    return pl.pallas_call(
        _fin_body,
        grid=(T // BT,),
        in_specs=[pl.BlockSpec((BT, H), lambda i: (i, 0)),
                  pl.BlockSpec((BT, H), lambda i: (i + T // BT, 0))],
        out_specs=pl.BlockSpec((BT,), lambda i: (i,)),
        out_shape=jax.ShapeDtypeStruct((T,), jnp.float32),
    )(d_all, d_all)


def kernel(entity_embeds, bases, coefficients, weights, relation_weight,
           edge_index, edge_type, masks, triple_batch):
    src = edge_index[0]
    dst = edge_index[1]
    npad_e = EPAD - E
    coff = jnp.arange(NC, dtype=jnp.int32)[:, None]
    gy = jnp.pad((masks * N + src).astype(jnp.int32), (0, npad_e))
    gy = (gy[None, :] + coff * (3 * N)).reshape(NC * NS * EGRP, EG, EB)
    gz = jnp.pad((masks * R + edge_type).astype(jnp.int32), (0, npad_e))
    gz = (gz[None, :] + coff * (3 * R)).reshape(NC * NS * EGRP, EG, EB)
    pad_sink = N + (jnp.arange(npad_e, dtype=jnp.int32) & (NP - N - 1))
    sd = jnp.concatenate([dst.astype(jnp.int32), pad_sink])
    sd = sd.reshape(NS * EGRP, EG, EB)
    hi = triple_batch[:, 0].astype(jnp.int32)
    hi = (hi[None, :] + coff * NP).reshape(NC * NS, TCH, TB)
    ri = triple_batch[:, 1].astype(jnp.int32)
    ri = (ri[None, :] + coff * R).reshape(NC * NS, TCH, TB)
    ti = triple_batch[:, 2].astype(jnp.int32)
    ti = (ti[None, :] + coff * NP).reshape(NC * NS, TCH, TB)

    z_tab, ur_tab = _rel_tables(coefficients, bases, weights, relation_weight)
    y_tab = _y_tables(entity_embeds, weights)
    x_all = _aggregate(y_tab, z_tab, gy, gz, sd)
    d_all = _score_gather(x_all, ur_tab, hi, ri, ti)
    return _finish(d_all)
